# baseline (speedup 1.0000x reference)
import functools

import jax
import jax.numpy as jnp
from jax import lax
from jax.experimental import pallas as pl
from jax.experimental.pallas import tpu as pltpu
from jax.experimental.pallas import tpu_sc as plsc

_B = 1024
_SEQ = 256
_DIM = 256

_NW = 32
_SPW = _SEQ // _NW
_BC = 64
_NCH = _B // _BC
_NPH = _SPW * _NCH


def _sc_body(x_hbm, pe_hbm, out_hbm, pe_v, in_a, in_b, out_a, out_b,
             sin_a, sin_b, sout_a, sout_b):
    wid = lax.axis_index("s") * 2 + lax.axis_index("c")
    s_base = wid * _SPW
    pltpu.sync_copy(pe_hbm.at[pl.ds(s_base, _SPW)], pe_v)

    def start_in(t, buf, sem):
        sl = lax.div(t, _NCH)
        b0 = lax.rem(t, _NCH) * _BC
        return pltpu.make_async_copy(
            x_hbm.at[pl.ds(b0, _BC), pl.ds(s_base + sl, 1)], buf, sem)

    def start_out(t, buf, sem):
        sl = lax.div(t, _NCH)
        b0 = lax.rem(t, _NCH) * _BC
        return pltpu.make_async_copy(
            buf, out_hbm.at[pl.ds(b0, _BC), pl.ds(s_base + sl, 1)], sem)

    def compute(t, src, dst):
        sl = lax.div(t, _NCH)

        def row(r, c):
            for j in range(_DIM // 16):
                d = pl.ds(j * 16, 16)
                dst[r, 0, d] = src[r, 0, d] + pe_v[sl, d]
            return c

        lax.fori_loop(0, _BC, row, 0)

    def phase(t, k, i_buf, o_buf, s_in, s_out, nxt_buf, s_nxt):
        @pl.when(t + 1 < _NPH)
        def _():
            start_in(t + 1, nxt_buf, s_nxt).start()

        pltpu.make_async_copy(x_hbm.at[pl.ds(0, _BC), pl.ds(0, 1)],
                              i_buf, s_in).wait()
        @pl.when(k > 0)
        def _():
            pltpu.make_async_copy(o_buf, out_hbm.at[pl.ds(0, _BC),
                                                    pl.ds(0, 1)], s_out).wait()

        compute(t, i_buf, o_buf)
        start_out(t, o_buf, s_out).start()

    start_in(0, in_a, sin_a).start()

    def body(k, carry):
        phase(2 * k, k, in_a, out_a, sin_a, sout_a, in_b, sin_b)
        phase(2 * k + 1, k, in_b, out_b, sin_b, sout_b, in_a, sin_a)
        return carry

    lax.fori_loop(0, _NPH // 2, body, 0)
    pltpu.make_async_copy(out_a, out_hbm.at[pl.ds(0, _BC), pl.ds(0, 1)],
                          sout_a).wait()
    pltpu.make_async_copy(out_b, out_hbm.at[pl.ds(0, _BC), pl.ds(0, 1)],
                          sout_b).wait()


def _sc_add(x, pe):
    kfn = functools.partial(
        pl.kernel,
        out_type=jax.ShapeDtypeStruct((_B, _SEQ, _DIM), jnp.float32),
        mesh=plsc.VectorSubcoreMesh(core_axis_name="c", subcore_axis_name="s"),
        scratch_types=[
            pltpu.VMEM((_SPW, _DIM), jnp.float32),
            pltpu.VMEM((_BC, 1, _DIM), jnp.float32),
            pltpu.VMEM((_BC, 1, _DIM), jnp.float32),
            pltpu.VMEM((_BC, 1, _DIM), jnp.float32),
            pltpu.VMEM((_BC, 1, _DIM), jnp.float32),
            pltpu.SemaphoreType.DMA,
            pltpu.SemaphoreType.DMA,
            pltpu.SemaphoreType.DMA,
            pltpu.SemaphoreType.DMA,
        ],
    )(_sc_body)
    return kfn(x, pe)


def kernel(x, pos_emb):
    B, S, D = x.shape
    pe = pos_emb[:S]
    return _sc_add(x, pe)


# speedup vs baseline: 3.3658x; 3.3658x over previous
import functools

import jax
import jax.numpy as jnp
from jax import lax
from jax.experimental import pallas as pl
from jax.experimental.pallas import tpu as pltpu
from jax.experimental.pallas import tpu_sc as plsc

_B = 1024
_SEQ = 256
_DIM = 256

_NBG = 8
_NQ = 4
_QROWS = _SEQ // _NQ
_BPG = _B // _NBG
_RC = 32
_CPB = _QROWS // _RC
_NCH = _BPG * _CPB
_DEPTH = 4


def _sc_body(x_hbm, pe_hbm, out_hbm, pe_v, *bufs_and_sems):
    ins = bufs_and_sems[0:_DEPTH]
    outs = bufs_and_sems[_DEPTH:2 * _DEPTH]
    sin = bufs_and_sems[2 * _DEPTH:3 * _DEPTH]
    sout = bufs_and_sems[3 * _DEPTH:4 * _DEPTH]

    wid = lax.axis_index("s") * 2 + lax.axis_index("c")
    bg = lax.div(wid, _NQ)
    q = lax.rem(wid, _NQ)
    pltpu.sync_copy(pe_hbm.at[pl.ds(q * _QROWS, _QROWS)], pe_v)

    def row0_of(t):
        b = bg * _BPG + lax.div(t, _CPB)
        return b * _SEQ + q * _QROWS + lax.rem(t, _CPB) * _RC

    def fire_in(t, i):
        pltpu.make_async_copy(
            x_hbm.at[pl.ds(row0_of(t), _RC)], ins[i], sin[i]).start()

    def fire_out(t, i):
        pltpu.make_async_copy(
            outs[i], out_hbm.at[pl.ds(row0_of(t), _RC)], sout[i]).start()

    def drain_in(i):
        pltpu.make_async_copy(
            x_hbm.at[pl.ds(0, _RC)], ins[i], sin[i]).wait()

    def drain_out(i):
        pltpu.make_async_copy(
            outs[i], out_hbm.at[pl.ds(0, _RC)], sout[i]).wait()

    def compute(t, i):
        p0 = lax.rem(t, _CPB) * _RC
        src, dst = ins[i], outs[i]

        def row(r, c):
            for j in range(_DIM // 16):
                d = pl.ds(j * 16, 16)
                dst[r, d] = src[r, d] + pe_v[p0 + r, d]
            return c

        lax.fori_loop(0, _RC, row, 0)

    for i in range(_DEPTH):
        fire_in(i, i)

    def body(k, carry):
        for i in range(_DEPTH):
            t = k * _DEPTH + i
            drain_in(i)

            @pl.when(k > 0)
            def _():
                drain_out(i)

            compute(t, i)
            fire_out(t, i)

            @pl.when(k < _NCH // _DEPTH - 1)
            def _():
                fire_in(t + _DEPTH, i)
        return carry

    lax.fori_loop(0, _NCH // _DEPTH, body, 0)
    for i in range(_DEPTH):
        drain_out(i)


def _sc_add(x2d, pe):
    kfn = functools.partial(
        pl.kernel,
        out_type=jax.ShapeDtypeStruct((_B * _SEQ, _DIM), jnp.float32),
        mesh=plsc.VectorSubcoreMesh(core_axis_name="c", subcore_axis_name="s"),
        scratch_types=(
            [pltpu.VMEM((_QROWS, _DIM), jnp.float32)]
            + [pltpu.VMEM((_RC, _DIM), jnp.float32) for _ in range(2 * _DEPTH)]
            + [pltpu.SemaphoreType.DMA for _ in range(2 * _DEPTH)]
        ),
    )(_sc_body)
    return kfn(x2d, pe)


def kernel(x, pos_emb):
    B, S, D = x.shape
    pe = pos_emb[:S]
    return _sc_add(x.reshape(B * S, D), pe).reshape(B, S, D)
